# baseline (device time: 11823 ns/iter reference)
import jax
import jax.numpy as jnp
from jax import lax
from jax.experimental import pallas as pl
from jax.experimental.pallas import tpu as pltpu

K = 8
IDX_MASK = 0x3FF
KEY_MIN = -(2**31)
N_CHUNK = 4

SORT8_NET = (
    (0, 1), (2, 3), (4, 5), (6, 7),
    (0, 2), (1, 3), (4, 6), (5, 7),
    (1, 2), (5, 6),
    (0, 4), (1, 5), (2, 6), (3, 7),
    (2, 4), (3, 5),
    (1, 2), (3, 4), (5, 6),
)


def kernel(x):
    m, n = x.shape
    dtype = x.dtype
    rows = m // N_CHUNK
    blocks = n // 128

    def _pack(vals):
        b = lax.bitcast_convert_type(vals, jnp.int32)
        s = jnp.where(b >= 0, b, b ^ 0x7FFFFFFF)
        iota = lax.broadcasted_iota(jnp.int32, (rows, n), 1)
        return (s & ~IDX_MASK) | iota

    def _unpack(keys):
        s = keys & ~IDX_MASK
        b = jnp.where(s >= 0, s, s ^ 0x7FFFFFFF)
        return lax.bitcast_convert_type(b, dtype)

    def _top8_desc_asc(keys):
        S = [keys[:, i * 128 : (i + 1) * 128] for i in range(blocks)]
        for a, b in SORT8_NET:
            hi = jnp.maximum(S[a], S[b])
            lo = jnp.minimum(S[a], S[b])
            S[a], S[b] = hi, lo
        out_iota = lax.broadcasted_iota(jnp.int32, (rows, K), 1)
        desc = jnp.full((rows, K), KEY_MIN, jnp.int32)
        asc = jnp.full((rows, K), KEY_MIN, jnp.int32)
        for k in range(K):
            mx = jnp.max(S[0], axis=1, keepdims=True)
            desc = jnp.where(out_iota == k, mx, desc)
            asc = jnp.where(out_iota == K - 1 - k, mx, asc)
            mask = S[0] == mx
            for i in range(blocks - 1):
                S[i] = jnp.where(mask, S[i + 1], S[i])
            S[blocks - 1] = jnp.where(mask, KEY_MIN, S[blocks - 1])
        return desc, asc

    def _merge_top8(desc_mine, asc_theirs):
        out = jnp.maximum(desc_mine, asc_theirs)
        iota8 = lax.broadcasted_iota(jnp.int32, (rows, K), 1)
        for d in (4, 2, 1):
            up = pltpu.roll(out, d, 1)
            down = pltpu.roll(out, K - d, 1)
            hi_lane = (iota8 & d) != 0
            partner = jnp.where(hi_lane, up, down)
            out = jnp.where(
                hi_lane, jnp.minimum(out, partner), jnp.maximum(out, partner)
            )
        return out

    def body(x_ref, out_ref, send_ref, recv_ref, send_sems, recv_sems):
        my_x = lax.axis_index("x")
        my_y = lax.axis_index("y")
        nbr = (my_x, 1 - my_y)

        barrier_sem = pltpu.get_barrier_semaphore()
        pl.semaphore_signal(
            barrier_sem, inc=1, device_id=nbr,
            device_id_type=pl.DeviceIdType.MESH,
        )

        def _swap(h):
            return pltpu.make_async_remote_copy(
                src_ref=send_ref.at[h],
                dst_ref=recv_ref.at[h],
                send_sem=send_sems.at[h],
                recv_sem=recv_sems.at[h],
                device_id=nbr,
                device_id_type=pl.DeviceIdType.MESH,
            )

        descs = []
        rdmas = []
        for h in range(N_CHUNK):
            desc, asc = _top8_desc_asc(_pack(x_ref[pl.ds(h * rows, rows), :]))
            descs.append(desc)
            send_ref[h, :, :] = asc
            if h == 0:
                pl.semaphore_wait(barrier_sem, 1)
            rdma = _swap(h)
            rdma.start()
            rdmas.append(rdma)

        for h in range(N_CHUNK):
            rdmas[h].wait()
            merged = _merge_top8(descs[h], recv_ref[h, :, :])
            out_ref[pl.ds(h * rows, rows), :] = _unpack(merged)

    return pl.pallas_call(
        body,
        out_shape=jax.ShapeDtypeStruct((m, K), dtype),
        in_specs=[pl.BlockSpec(memory_space=pltpu.VMEM)],
        out_specs=pl.BlockSpec(memory_space=pltpu.VMEM),
        scratch_shapes=[
            pltpu.VMEM((N_CHUNK, rows, K), jnp.int32),
            pltpu.VMEM((N_CHUNK, rows, K), jnp.int32),
            pltpu.SemaphoreType.DMA((N_CHUNK,)),
            pltpu.SemaphoreType.DMA((N_CHUNK,)),
        ],
        compiler_params=pltpu.CompilerParams(collective_id=0),
    )(x)


# device time: 9637 ns/iter; 1.2268x vs baseline; 1.2268x over previous
import jax
import jax.numpy as jnp
from jax import lax
from jax.experimental import pallas as pl
from jax.experimental.pallas import tpu as pltpu

K = 8
IDX_MASK = 0x3FF
KEY_MIN = -(2**31)
N_CHUNK = 2

SORT8_NET = (
    (0, 1), (2, 3), (4, 5), (6, 7),
    (0, 2), (1, 3), (4, 6), (5, 7),
    (1, 2), (5, 6),
    (0, 4), (1, 5), (2, 6), (3, 7),
    (2, 4), (3, 5),
    (1, 2), (3, 4), (5, 6),
)


def kernel(x):
    m, n = x.shape
    dtype = x.dtype
    rows = m // N_CHUNK
    blocks = n // 128

    def _pack(vals):
        b = lax.bitcast_convert_type(vals, jnp.int32)
        s = jnp.where(b >= 0, b, b ^ 0x7FFFFFFF)
        iota = lax.broadcasted_iota(jnp.int32, (rows, n), 1)
        return (s & ~IDX_MASK) | iota

    def _unpack(keys):
        s = keys & ~IDX_MASK
        b = jnp.where(s >= 0, s, s ^ 0x7FFFFFFF)
        return lax.bitcast_convert_type(b, dtype)

    def _top8_desc_asc(keys):
        S = [keys[:, i * 128 : (i + 1) * 128] for i in range(blocks)]
        for a, b in SORT8_NET:
            hi = jnp.maximum(S[a], S[b])
            lo = jnp.minimum(S[a], S[b])
            S[a], S[b] = hi, lo
        out_iota = lax.broadcasted_iota(jnp.int32, (rows, K), 1)
        desc = jnp.full((rows, K), KEY_MIN, jnp.int32)
        asc = jnp.full((rows, K), KEY_MIN, jnp.int32)
        for k in range(K):
            mx = jnp.max(S[0], axis=1, keepdims=True)
            desc = jnp.where(out_iota == k, mx, desc)
            asc = jnp.where(out_iota == K - 1 - k, mx, asc)
            mask = S[0] == mx
            for i in range(blocks - 1):
                S[i] = jnp.where(mask, S[i + 1], S[i])
            S[blocks - 1] = jnp.where(mask, KEY_MIN, S[blocks - 1])
        return desc, asc

    def _merge_top8(desc_mine, asc_theirs):
        out = jnp.maximum(desc_mine, asc_theirs)
        iota8 = lax.broadcasted_iota(jnp.int32, (rows, K), 1)
        for d in (4, 2, 1):
            up = pltpu.roll(out, d, 1)
            down = pltpu.roll(out, K - d, 1)
            hi_lane = (iota8 & d) != 0
            partner = jnp.where(hi_lane, up, down)
            out = jnp.where(
                hi_lane, jnp.minimum(out, partner), jnp.maximum(out, partner)
            )
        return out

    def body(x_ref, out_ref, send_ref, recv_ref, send_sems, recv_sems):
        my_x = lax.axis_index("x")
        my_y = lax.axis_index("y")
        nbr = (my_x, 1 - my_y)

        barrier_sem = pltpu.get_barrier_semaphore()
        pl.semaphore_signal(
            barrier_sem, inc=1, device_id=nbr,
            device_id_type=pl.DeviceIdType.MESH,
        )

        def _swap(h):
            return pltpu.make_async_remote_copy(
                src_ref=send_ref.at[h],
                dst_ref=recv_ref.at[h],
                send_sem=send_sems.at[h],
                recv_sem=recv_sems.at[h],
                device_id=nbr,
                device_id_type=pl.DeviceIdType.MESH,
            )

        descs = []
        rdmas = []
        for h in range(N_CHUNK):
            desc, asc = _top8_desc_asc(_pack(x_ref[pl.ds(h * rows, rows), :]))
            descs.append(desc)
            send_ref[h, :, :] = asc
            if h == 0:
                pl.semaphore_wait(barrier_sem, 1)
            rdma = _swap(h)
            rdma.start()
            rdmas.append(rdma)

        for h in range(N_CHUNK):
            rdmas[h].wait()
            merged = _merge_top8(descs[h], recv_ref[h, :, :])
            out_ref[pl.ds(h * rows, rows), :] = _unpack(merged)

    return pl.pallas_call(
        body,
        out_shape=jax.ShapeDtypeStruct((m, K), dtype),
        in_specs=[pl.BlockSpec(memory_space=pltpu.VMEM)],
        out_specs=pl.BlockSpec(memory_space=pltpu.VMEM),
        scratch_shapes=[
            pltpu.VMEM((N_CHUNK, rows, K), jnp.int32),
            pltpu.VMEM((N_CHUNK, rows, K), jnp.int32),
            pltpu.SemaphoreType.DMA((N_CHUNK,)),
            pltpu.SemaphoreType.DMA((N_CHUNK,)),
        ],
        compiler_params=pltpu.CompilerParams(collective_id=0),
    )(x)


# device time: 9363 ns/iter; 1.2627x vs baseline; 1.0293x over previous
import jax
import jax.numpy as jnp
from jax import lax
from jax.experimental import pallas as pl
from jax.experimental.pallas import tpu as pltpu

K = 8
IDX_MASK = 0x3FF
KEY_MIN = -(2**31)
N_CHUNK = 1

SORT8_NET = (
    (0, 1), (2, 3), (4, 5), (6, 7),
    (0, 2), (1, 3), (4, 6), (5, 7),
    (1, 2), (5, 6),
    (0, 4), (1, 5), (2, 6), (3, 7),
    (2, 4), (3, 5),
    (1, 2), (3, 4), (5, 6),
)


def kernel(x):
    m, n = x.shape
    dtype = x.dtype
    rows = m // N_CHUNK
    blocks = n // 128

    def _pack(vals):
        b = lax.bitcast_convert_type(vals, jnp.int32)
        s = jnp.where(b >= 0, b, b ^ 0x7FFFFFFF)
        iota = lax.broadcasted_iota(jnp.int32, (rows, n), 1)
        return (s & ~IDX_MASK) | iota

    def _unpack(keys):
        s = keys & ~IDX_MASK
        b = jnp.where(s >= 0, s, s ^ 0x7FFFFFFF)
        return lax.bitcast_convert_type(b, dtype)

    def _top8_desc_asc(keys):
        S = [keys[:, i * 128 : (i + 1) * 128] for i in range(blocks)]
        for a, b in SORT8_NET:
            hi = jnp.maximum(S[a], S[b])
            lo = jnp.minimum(S[a], S[b])
            S[a], S[b] = hi, lo
        out_iota = lax.broadcasted_iota(jnp.int32, (rows, K), 1)
        desc = jnp.full((rows, K), KEY_MIN, jnp.int32)
        asc = jnp.full((rows, K), KEY_MIN, jnp.int32)
        for k in range(K):
            mx = jnp.max(S[0], axis=1, keepdims=True)
            desc = jnp.where(out_iota == k, mx, desc)
            asc = jnp.where(out_iota == K - 1 - k, mx, asc)
            mask = S[0] == mx
            for i in range(blocks - 1):
                S[i] = jnp.where(mask, S[i + 1], S[i])
            S[blocks - 1] = jnp.where(mask, KEY_MIN, S[blocks - 1])
        return desc, asc

    def _merge_top8(desc_mine, asc_theirs):
        out = jnp.maximum(desc_mine, asc_theirs)
        iota8 = lax.broadcasted_iota(jnp.int32, (rows, K), 1)
        for d in (4, 2, 1):
            up = pltpu.roll(out, d, 1)
            down = pltpu.roll(out, K - d, 1)
            hi_lane = (iota8 & d) != 0
            partner = jnp.where(hi_lane, up, down)
            out = jnp.where(
                hi_lane, jnp.minimum(out, partner), jnp.maximum(out, partner)
            )
        return out

    def body(x_ref, out_ref, send_ref, recv_ref, send_sems, recv_sems):
        my_x = lax.axis_index("x")
        my_y = lax.axis_index("y")
        nbr = (my_x, 1 - my_y)

        barrier_sem = pltpu.get_barrier_semaphore()
        pl.semaphore_signal(
            barrier_sem, inc=1, device_id=nbr,
            device_id_type=pl.DeviceIdType.MESH,
        )

        def _swap(h):
            return pltpu.make_async_remote_copy(
                src_ref=send_ref.at[h],
                dst_ref=recv_ref.at[h],
                send_sem=send_sems.at[h],
                recv_sem=recv_sems.at[h],
                device_id=nbr,
                device_id_type=pl.DeviceIdType.MESH,
            )

        descs = []
        rdmas = []
        for h in range(N_CHUNK):
            desc, asc = _top8_desc_asc(_pack(x_ref[pl.ds(h * rows, rows), :]))
            descs.append(desc)
            send_ref[h, :, :] = asc
            if h == 0:
                pl.semaphore_wait(barrier_sem, 1)
            rdma = _swap(h)
            rdma.start()
            rdmas.append(rdma)

        for h in range(N_CHUNK):
            rdmas[h].wait()
            merged = _merge_top8(descs[h], recv_ref[h, :, :])
            out_ref[pl.ds(h * rows, rows), :] = _unpack(merged)

    return pl.pallas_call(
        body,
        out_shape=jax.ShapeDtypeStruct((m, K), dtype),
        in_specs=[pl.BlockSpec(memory_space=pltpu.VMEM)],
        out_specs=pl.BlockSpec(memory_space=pltpu.VMEM),
        scratch_shapes=[
            pltpu.VMEM((N_CHUNK, rows, K), jnp.int32),
            pltpu.VMEM((N_CHUNK, rows, K), jnp.int32),
            pltpu.SemaphoreType.DMA((N_CHUNK,)),
            pltpu.SemaphoreType.DMA((N_CHUNK,)),
        ],
        compiler_params=pltpu.CompilerParams(collective_id=0),
    )(x)
